# Initial kernel scaffold; baseline (speedup 1.0000x reference)
import jax
import jax.numpy as jnp
from jax.experimental import pallas as pl

B, L, HIDDEN = 2, 2048, 1024
HEADS, HEAD_DIM = 16, 64
CLUSTERS, ITERS = 128, 2


def _fused_body(x_ref, maskr_ref, maskc_ref, p_ref,
                wq_ref, bq_ref, wk_ref, bk_ref, wv_ref, bv_ref,
                wo_ref, bo_ref, y_ref):
    h = pl.program_id(1)
    f32 = jnp.float32
    x = x_ref[0]
    q = jnp.dot(x, wq_ref[...], preferred_element_type=f32) + bq_ref[...]
    k = jnp.dot(x, wk_ref[...], preferred_element_type=f32) + bk_ref[...]
    v = jnp.dot(x, wv_ref[...], preferred_element_type=f32) + bv_ref[...]
    maskc = maskc_ref[0]
    maskr = maskr_ref[0]

    cent = jnp.dot(p_ref[...], q, preferred_element_type=f32)
    qsq = jnp.sum(q * q, axis=1, keepdims=True)
    iota_c = jax.lax.broadcasted_iota(jnp.int32, (L, CLUSTERS), 1)
    ones_col = jnp.ones((L, 1), f32)
    grp = None
    onehot = None
    for _ in range(ITERS):
        centsq = jnp.sum(cent * cent, axis=1)
        qc = jax.lax.dot_general(q, cent, (((1,), (1,)), ((), ())),
                                 preferred_element_type=f32)
        d = qsq - 2.0 * qc + centsq.reshape(1, CLUSTERS)
        dmin = jnp.min(d, axis=1, keepdims=True)
        grp = jnp.min(jnp.where(d == dmin, iota_c, CLUSTERS),
                      axis=1, keepdims=True)
        onehot = jnp.where(iota_c == grp, 1.0, 0.0).astype(f32) * maskc
        counts = jax.lax.dot_general(onehot, ones_col, (((0,), (0,)), ((), ())),
                                     preferred_element_type=f32)
        sums = jax.lax.dot_general(onehot, q, (((0,), (0,)), ((), ())),
                                   preferred_element_type=f32)
        new_cent = sums / jnp.maximum(counts, 1.0)
        cent = jnp.where(counts > 0, new_cent, cent)

    scale = f32(1.0 / (HEAD_DIM ** 0.5))
    logits = jax.lax.dot_general(cent, k, (((1,), (1,)), ((), ())),
                                 preferred_element_type=f32) * scale
    logits = jnp.where(maskr > 0.0, logits, f32(-1e9))
    mx = jnp.max(logits, axis=1, keepdims=True)
    e = jnp.exp(logits - mx)
    a = e / jnp.sum(e, axis=1, keepdims=True)
    out_c = jnp.dot(a, v, preferred_element_type=f32)

    outh = jnp.dot(onehot, out_c, preferred_element_type=f32)
    part = jnp.dot(outh, wo_ref[...], preferred_element_type=f32)

    @pl.when(h == 0)
    def _():
        y_ref[0] = part + bo_ref[...]

    @pl.when(h != 0)
    def _():
        y_ref[0] = y_ref[0] + part


def kernel(X, attn_mask, length_mask, Wq, bq, Wk, bk, Wv, bv, Wo, bo):
    f32 = jnp.float32
    pos = jnp.arange(L, dtype=jnp.int32)
    maskf = (attn_mask & (pos[None, :] < length_mask[:, None])).astype(f32)
    maskr = maskf.reshape(B, 1, L)
    maskc = maskf.reshape(B, L, 1)
    init_idx = jnp.linspace(0, L - 1, CLUSTERS).astype(jnp.int32)
    p_init = jax.nn.one_hot(init_idx, L, dtype=f32)

    grid = (B, HEADS)
    out = pl.pallas_call(
        _fused_body,
        grid=grid,
        in_specs=[
            pl.BlockSpec((1, L, HIDDEN), lambda b, h: (b, 0, 0)),
            pl.BlockSpec((1, 1, L), lambda b, h: (b, 0, 0)),
            pl.BlockSpec((1, L, 1), lambda b, h: (b, 0, 0)),
            pl.BlockSpec((CLUSTERS, L), lambda b, h: (0, 0)),
            pl.BlockSpec((HIDDEN, HEAD_DIM), lambda b, h: (0, h)),
            pl.BlockSpec((1, HEAD_DIM), lambda b, h: (0, h)),
            pl.BlockSpec((HIDDEN, HEAD_DIM), lambda b, h: (0, h)),
            pl.BlockSpec((1, HEAD_DIM), lambda b, h: (0, h)),
            pl.BlockSpec((HIDDEN, HEAD_DIM), lambda b, h: (0, h)),
            pl.BlockSpec((1, HEAD_DIM), lambda b, h: (0, h)),
            pl.BlockSpec((HEAD_DIM, HIDDEN), lambda b, h: (h, 0)),
            pl.BlockSpec((1, HIDDEN), lambda b, h: (0, 0)),
        ],
        out_specs=pl.BlockSpec((1, L, HIDDEN), lambda b, h: (b, 0, 0)),
        out_shape=jax.ShapeDtypeStruct((B, L, HIDDEN), f32),
    )(X, maskr, maskc, p_init,
      Wq, bq.reshape(1, -1), Wk, bk.reshape(1, -1), Wv, bv.reshape(1, -1),
      Wo, bo.reshape(1, -1))
    return out


# speedup vs baseline: 138.1891x; 138.1891x over previous
import jax
import jax.numpy as jnp
from jax.experimental import pallas as pl

B, L, HIDDEN = 2, 2048, 1024
HEADS, HEAD_DIM = 16, 64
CLUSTERS, ITERS = 128, 2

_f32 = jnp.float32


def _mm_seq(a, w, chunk=256):
    k = a.shape[1]
    acc = jnp.dot(a[:, 0:chunk], w[0:chunk], preferred_element_type=_f32)
    for i in range(1, k // chunk):
        acc = acc + jnp.dot(a[:, chunk * i:chunk * (i + 1)],
                            w[chunk * i:chunk * (i + 1)],
                            preferred_element_type=_f32)
    return acc


def _mm_t_seq(a, bmat, chunk=256):
    k = a.shape[0]
    dn = (((0,), (0,)), ((), ()))
    acc = jax.lax.dot_general(a[0:chunk], bmat[0:chunk], dn,
                              preferred_element_type=_f32)
    for i in range(1, k // chunk):
        acc = acc + jax.lax.dot_general(a[chunk * i:chunk * (i + 1)],
                                        bmat[chunk * i:chunk * (i + 1)], dn,
                                        preferred_element_type=_f32)
    return acc


def _fused_body(x_ref, xi_ref, maskr_ref, maskc_ref,
                wq_ref, bq_ref, wk_ref, bk_ref, wv_ref, bv_ref,
                wo_ref, bo_ref, y_ref):
    h = pl.program_id(1)
    x = x_ref[0]
    q = _mm_seq(x, wq_ref[0]) + bq_ref[0]
    k = _mm_seq(x, wk_ref[0]) + bk_ref[0]
    v = _mm_seq(x, wv_ref[0]) + bv_ref[0]
    maskc = maskc_ref[0]
    maskr = maskr_ref[0]

    cent = _mm_seq(xi_ref[0], wq_ref[0]) + bq_ref[0]
    qsq = jnp.sum(q * q, axis=1, keepdims=True)
    iota_c = jax.lax.broadcasted_iota(jnp.int32, (L, CLUSTERS), 1)
    ones_col = jnp.ones((L, 1), _f32)
    grp = None
    onehot = None
    for _ in range(ITERS):
        centsq = jnp.sum(cent * cent, axis=1)
        qc = jax.lax.dot_general(q, cent, (((1,), (1,)), ((), ())),
                                 preferred_element_type=_f32)
        d = qsq - 2.0 * qc + centsq.reshape(1, CLUSTERS)
        dmin = jnp.min(d, axis=1, keepdims=True)
        grp = jnp.min(jnp.where(d == dmin, iota_c, CLUSTERS),
                      axis=1, keepdims=True)
        onehot = jnp.where(iota_c == grp, 1.0, 0.0).astype(_f32) * maskc
        counts = jax.lax.dot_general(onehot, ones_col, (((0,), (0,)), ((), ())),
                                     preferred_element_type=_f32)
        sums = _mm_t_seq(onehot, q)
        new_cent = sums / jnp.maximum(counts, 1.0)
        cent = jnp.where(counts > 0, new_cent, cent)

    scale = _f32(1.0 / (HEAD_DIM ** 0.5))
    logits = jax.lax.dot_general(cent, k, (((1,), (1,)), ((), ())),
                                 preferred_element_type=_f32) * scale
    logits = jnp.where(maskr > 0.0, logits, _f32(-1e9))
    mx = jnp.max(logits, axis=1, keepdims=True)
    e = jnp.exp(logits - mx)
    a = e / jnp.sum(e, axis=1, keepdims=True)
    out_c = jnp.dot(a, v, preferred_element_type=_f32)

    outh = jnp.dot(onehot, out_c, preferred_element_type=_f32)
    part = jnp.dot(outh, wo_ref[0], preferred_element_type=_f32)

    @pl.when(h == 0)
    def _():
        y_ref[0] = part + bo_ref[...]

    @pl.when(h != 0)
    def _():
        y_ref[0] = y_ref[0] + part


def kernel(X, attn_mask, length_mask, Wq, bq, Wk, bk, Wv, bv, Wo, bo):
    pos = jnp.arange(L, dtype=jnp.int32)
    maskf = (attn_mask & (pos[None, :] < length_mask[:, None])).astype(_f32)
    maskr = maskf.reshape(B, 1, L)
    maskc = maskf.reshape(B, L, 1)
    init_idx = jnp.linspace(0, L - 1, CLUSTERS).astype(jnp.int32)
    xinit = X[:, init_idx, :]

    wq3 = Wq.reshape(HIDDEN, HEADS, HEAD_DIM).transpose(1, 0, 2)
    wk3 = Wk.reshape(HIDDEN, HEADS, HEAD_DIM).transpose(1, 0, 2)
    wv3 = Wv.reshape(HIDDEN, HEADS, HEAD_DIM).transpose(1, 0, 2)
    wo3 = Wo.reshape(HEADS, HEAD_DIM, HIDDEN)

    grid = (B, HEADS)
    out = pl.pallas_call(
        _fused_body,
        grid=grid,
        in_specs=[
            pl.BlockSpec((1, L, HIDDEN), lambda b, h: (b, 0, 0)),
            pl.BlockSpec((1, CLUSTERS, HIDDEN), lambda b, h: (b, 0, 0)),
            pl.BlockSpec((1, 1, L), lambda b, h: (b, 0, 0)),
            pl.BlockSpec((1, L, 1), lambda b, h: (b, 0, 0)),
            pl.BlockSpec((1, HIDDEN, HEAD_DIM), lambda b, h: (h, 0, 0)),
            pl.BlockSpec((1, 1, HEAD_DIM), lambda b, h: (h, 0, 0)),
            pl.BlockSpec((1, HIDDEN, HEAD_DIM), lambda b, h: (h, 0, 0)),
            pl.BlockSpec((1, 1, HEAD_DIM), lambda b, h: (h, 0, 0)),
            pl.BlockSpec((1, HIDDEN, HEAD_DIM), lambda b, h: (h, 0, 0)),
            pl.BlockSpec((1, 1, HEAD_DIM), lambda b, h: (h, 0, 0)),
            pl.BlockSpec((1, HEAD_DIM, HIDDEN), lambda b, h: (h, 0, 0)),
            pl.BlockSpec((1, HIDDEN), lambda b, h: (0, 0)),
        ],
        out_specs=pl.BlockSpec((1, L, HIDDEN), lambda b, h: (b, 0, 0)),
        out_shape=jax.ShapeDtypeStruct((B, L, HIDDEN), _f32),
    )(X, xinit, maskr, maskc,
      wq3, bq.reshape(HEADS, 1, HEAD_DIM), wk3, bk.reshape(HEADS, 1, HEAD_DIM),
      wv3, bv.reshape(HEADS, 1, HEAD_DIM), wo3, bo.reshape(1, -1))
    return out
